# baseline (speedup 1.0000x reference)
import functools

import jax
import jax.numpy as jnp
import numpy as np
from jax import lax
from jax.experimental import pallas as pl
from jax.experimental.pallas import tpu as pltpu
from jax.experimental.pallas import tpu_sc as plsc

N = 10000
DEG = 32
D = 128
L1 = 256
L2 = 128
S = 25
B = 1024
BS = B * S
M = BS + B

NW = 32
RPW = BS // NW
NPW = B // NW
CR = 16
NCH = RPW // CR
CI = CR * S
CH = CI // 2
GG = 40
NG = CH // GG


def _tf_cipher(k1, k2, x0, x1):
    ks = [np.uint32(k1), np.uint32(k2), np.uint32(k1 ^ k2 ^ 0x1BD11BDA)]
    rots = [[13, 15, 26, 6], [17, 29, 16, 24]]
    x0 = (x0 + ks[0]).astype(np.uint32)
    x1 = (x1 + ks[1]).astype(np.uint32)
    for i in range(5):
        for d in rots[i % 2]:
            x0 = (x0 + x1).astype(np.uint32)
            x1 = ((x1 << np.uint32(d)) | (x1 >> np.uint32(32 - d))).astype(
                np.uint32
            )
            x1 = (x1 ^ x0).astype(np.uint32)
        x0 = (x0 + ks[(i + 1) % 3]).astype(np.uint32)
        x1 = (x1 + ks[(i + 2) % 3] + np.uint32(i + 1)).astype(np.uint32)
    return x0, x1


def _tf_bits(kd, size):
    o0, o1 = _tf_cipher(
        kd[0], kd[1], np.zeros(size, np.uint32),
        np.arange(size, dtype=np.uint32),
    )
    return (o0 ^ o1).astype(np.uint32)


def _tf_randint(kd, shape, maxval):
    size = int(np.prod(shape))
    o0, o1 = _tf_cipher(
        kd[0], kd[1], np.zeros(2, np.uint32), np.arange(2, dtype=np.uint32)
    )
    hi = _tf_bits((o0[0], o1[0]), size)
    lo = _tf_bits((o0[1], o1[1]), size)
    span = np.uint32(maxval)
    mult = np.uint32(((2**16 % maxval) ** 2) % maxval)
    off = ((hi % span) * mult + (lo % span)) % span
    return off.astype(np.int32).reshape(shape)


_s0, _s1 = _tf_cipher(
    np.uint32(0), np.uint32(42), np.zeros(2, np.uint32),
    np.arange(2, dtype=np.uint32),
)
_C2 = _tf_randint((_s0[0], _s1[0]), (B, S), DEG)
_C1 = _tf_randint((_s0[1], _s1[1]), (M, S), DEG)[:BS]
_C2R = _C2.reshape(NW, RPW // 16, 16)
_C1R = _C1.reshape(NW * NCH, CI // 16, 16)


def _sc_gather_body(adj_h, feat_h, nodes_h, c2_h, c1_h, agg1_h, subf_h,
                    nodes_v, adja_v, c2_v, sn_v, adjb0_v, adjb1_v, c10_v,
                    c11_v, idx0_v, idx1_v, fbuf0_v, fbuf1_v, acc_v, sf_v,
                    sem_a, sem_sf, sem_b0, sem_b1, sem_c10, sem_c11,
                    sem_g0, sem_g1):
    w = lax.axis_index("s") * 2 + lax.axis_index("c")
    inv_s = jnp.full((16,), 1.0 / S, dtype=jnp.float32)
    adjb = (adjb0_v, adjb1_v)
    c1b = (c10_v, c11_v)
    idxb = (idx0_v, idx1_v)
    fbuf = (fbuf0_v, fbuf1_v)
    sem_b = (sem_b0, sem_b1)
    sem_c1 = (sem_c10, sem_c11)
    sem_g = (sem_g0, sem_g1)

    pltpu.sync_copy(nodes_h.at[pl.ds(w * NPW, NPW)], nodes_v)
    cp_a = pltpu.async_copy(adj_h.at[nodes_v], adja_v, sem_a)
    cp_sf = pltpu.async_copy(feat_h.at[nodes_v], sf_v, sem_sf)
    pltpu.sync_copy(c2_h.at[w], c2_v)
    cp_a.wait()

    def sn_step(i, _):
        p = lax.iota(jnp.int32, 16) + i * 16
        rows = lax.shift_right_logical(p * 5243, 17)
        cols = c2_v[i, :]
        sn_v[i, :] = plsc.load_gather(adja_v, [rows, cols])
        return 0

    lax.fori_loop(0, RPW // 16, sn_step, 0)
    cp_sf.wait()
    pltpu.sync_copy(sf_v, subf_h.at[pl.ds(w * NPW, NPW)])

    def issue_pre(c, p):
        pltpu.async_copy(adj_h.at[sn_v.at[c]], adjb[p], sem_b[p])
        pltpu.async_copy(c1_h.at[w * NCH + c], c1b[p], sem_c1[p])

    def wait_pre(c, p):
        pltpu.make_async_copy(adj_h.at[sn_v.at[c]], adjb[p], sem_b[p]).wait()
        pltpu.make_async_copy(c1_h.at[w * NCH + c], c1b[p], sem_c1[p]).wait()

    def compute_idx(p):
        def idx_step(i, _):
            q = lax.iota(jnp.int32, 16) + i * 16
            rows = lax.shift_right_logical(q * 5243, 17)
            cols = c1b[p][i, :]
            idxb[p][pl.ds(i * 16, 16)] = plsc.load_gather(adjb[p], [rows, cols])
            return 0

        lax.fori_loop(0, CI // 16, idx_step, 0)

    def fire_half(ip, h):
        for j in range(NG):
            pltpu.async_copy(
                feat_h.at[idxb[ip].at[pl.ds(h * CH + j * GG, GG)]],
                fbuf[h].at[pl.ds(j * GG, GG)],
                sem_g[h],
            )

    def drain_half(h):
        pltpu.make_async_copy(feat_h.at[pl.ds(0, CH)], fbuf[h], sem_g[h]).wait()

    def accumulate_half(h, row0):
        fb = fbuf[h]
        fmt = plsc.PackFormat.INTERLEAVED

        def ldrow(row, k):
            v = plsc.bitcast(fb[row, pl.ds(k * 16, 16)], jnp.bfloat16)
            return plsc.unpack(v, format=fmt)

        def row_step(r, _):
            base = r * S
            acc = [list(ldrow(base, k)) for k in range(4)]
            for s in range(1, S):
                for k in range(4):
                    e, o = ldrow(base + s, k)
                    acc[k][0] = acc[k][0] + e
                    acc[k][1] = acc[k][1] + o
            for k in range(4):
                acc_v[row0 + r, pl.ds(k * 32, 32)] = plsc.pack(
                    acc[k][0] * inv_s, acc[k][1] * inv_s, format=fmt
                )
            return 0

        lax.fori_loop(0, CR // 2, row_step, 0)

    pltpu.sync_copy(c1_h.at[w * NCH], c10_v)
    pltpu.async_copy(adj_h.at[sn_v.at[0]], adjb0_v, sem_b0).wait()
    compute_idx(0)
    fire_half(0, 0)
    issue_pre(1, 1)

    def pair(t, _):
        for pc in (0, 1):
            c = 2 * t + pc
            nxt, pp = c + 1, 1 - pc
            drain_half(0)
            fire_half(pc, 1)
            if pc == 0:
                wait_pre(nxt, pp)
                compute_idx(pp)

                @pl.when(t < (NCH // 2) - 1)
                def _():
                    issue_pre(nxt + 1, pc)
            else:

                @pl.when(t < (NCH // 2) - 1)
                def _():
                    wait_pre(nxt, pp)
                    compute_idx(pp)
                    issue_pre(nxt + 1, pc)

            accumulate_half(0, 0)
            drain_half(1)
            if pc == 0:
                fire_half(pp, 0)
            else:

                @pl.when(t < (NCH // 2) - 1)
                def _():
                    fire_half(pp, 0)

            accumulate_half(1, CR // 2)
            pltpu.sync_copy(acc_v, agg1_h.at[pl.ds(w * RPW + c * CR, CR)])
        return 0

    lax.fori_loop(0, NCH // 2, pair, 0)


@functools.partial(jax.jit, static_argnums=())
def _sc_gather(adj, features, nodes, c2r, c1r):
    mesh = plsc.VectorSubcoreMesh(core_axis_name="c", subcore_axis_name="s")
    return pl.kernel(
        _sc_gather_body,
        out_type=[
            jax.ShapeDtypeStruct((BS, D), jnp.bfloat16),
            jax.ShapeDtypeStruct((B, D), jnp.int32),
        ],
        mesh=mesh,
        compiler_params=pltpu.CompilerParams(needs_layout_passes=False),
        scratch_types=[
            pltpu.VMEM((NPW,), jnp.int32),
            pltpu.VMEM((NPW, 128), jnp.int32),
            pltpu.VMEM((RPW // 16, 16), jnp.int32),
            pltpu.VMEM((NCH, CR), jnp.int32),
            pltpu.VMEM((CR, 128), jnp.int32),
            pltpu.VMEM((CR, 128), jnp.int32),
            pltpu.VMEM((CI // 16, 16), jnp.int32),
            pltpu.VMEM((CI // 16, 16), jnp.int32),
            pltpu.VMEM((CI,), jnp.int32),
            pltpu.VMEM((CI,), jnp.int32),
            pltpu.VMEM((CH, D), jnp.int32),
            pltpu.VMEM((CH, D), jnp.int32),
            pltpu.VMEM((CR, D), jnp.bfloat16),
            pltpu.VMEM((NPW, D), jnp.int32),
            pltpu.SemaphoreType.DMA,
            pltpu.SemaphoreType.DMA,
            pltpu.SemaphoreType.DMA,
            pltpu.SemaphoreType.DMA,
            pltpu.SemaphoreType.DMA,
            pltpu.SemaphoreType.DMA,
            pltpu.SemaphoreType.DMA,
            pltpu.SemaphoreType.DMA,
        ],
    )(adj, features, nodes, c2r, c1r)


_R1 = 800
_G1 = _R1 // S
_HI = jax.lax.Precision.HIGHEST


def _l2n(x):
    n = jnp.sqrt(jnp.sum(x * x, axis=-1, keepdims=True))
    return x / jnp.maximum(n, 1e-12)


def _dot(a, b):
    return jnp.dot(a.astype(jnp.bfloat16), b.astype(jnp.bfloat16),
                   preferred_element_type=jnp.float32)


def _dot_hi(a, b):
    return jnp.dot(a, b, preferred_element_type=jnp.float32, precision=_HI)


def _dot_nt(a, b):
    return lax.dot_general(
        a.astype(jnp.bfloat16), b.astype(jnp.bfloat16),
        (((1,), (1,)), ((), ())), preferred_element_type=jnp.float32,
    )


def _stage1_body(agg1_ref, w1_ref, out_ref):
    x = jnp.maximum(_dot(agg1_ref[...], w1_ref[...]), 0.0)
    row = lax.broadcasted_iota(jnp.int32, (_G1, _R1), 0)
    col = lax.broadcasted_iota(jnp.int32, (_G1, _R1), 1)
    g = jnp.where(col // S == row, 1.0 / S, 0.0).astype(jnp.float32)
    out_ref[...] = _dot_hi(g, x)


def _stage2_body(agg2_ref, w2_ref, w3_ref, wd1_ref, wd2_ref, wm1_ref,
                 wm2_ref, wg_ref, wl_ref, sub_adj_ref, sub_feat_ref,
                 mu_ref, logv_ref, rec_ref, pred_ref):
    a = agg2_ref[...]
    mu = _dot(a, w2_ref[...])
    mu_ref[...] = mu
    logv_ref[...] = -_dot(a, w3_ref[...])
    h = _l2n(mu)
    o = _dot(h, wd1_ref[...])
    o = jnp.maximum(_l2n(o), 0.0)
    o = _dot(o, wd2_ref[...])
    o = jnp.maximum(_l2n(o), 0.0)
    e1 = _l2n(_dot(o, wm1_ref[...]))
    e2 = _l2n(_dot(o, wm2_ref[...]))
    rec = _dot_nt(e1, e2)
    rec_ref[...] = rec
    sg = _dot(sub_feat_ref[...], wg_ref[...])
    og = _dot(sub_adj_ref[...], sg)
    orig = _dot(og, wl_ref[...])
    ge = _dot(rec, sg)
    gen = _dot(ge, wl_ref[...])
    pred_ref[...] = jnp.concatenate([orig, gen], axis=0)


def kernel(nodes, sub_adj, adj, features, W1, W2, W3, Wd1, Wd2, Wm1, Wm2,
           Wg, Wl):
    nodes = nodes.astype(jnp.int32)
    adj_p = jnp.pad(adj.astype(jnp.int32), ((0, 0), (0, 128 - DEG)))
    f16 = features.astype(jnp.bfloat16).reshape(N, D // 2, 2)
    fpk = jnp.pad(jax.lax.bitcast_convert_type(f16, jnp.int32),
                  ((0, 0), (0, D // 2)))
    agg1, sub_feat_pk = _sc_gather(
        adj_p, fpk, nodes, jnp.asarray(_C2R), jnp.asarray(_C1R)
    )
    sub_feat = jax.lax.bitcast_convert_type(
        sub_feat_pk[:, : D // 2], jnp.bfloat16
    ).reshape(B, D).astype(jnp.float32)

    agg2 = pl.pallas_call(
        _stage1_body,
        grid=(BS // _R1,),
        in_specs=[
            pl.BlockSpec((_R1, D), lambda i: (i, 0)),
            pl.BlockSpec((D, L1), lambda i: (0, 0)),
        ],
        out_specs=pl.BlockSpec((_G1, L1), lambda i: (i, 0)),
        out_shape=jax.ShapeDtypeStruct((B, L1), jnp.float32),
    )(agg1, W1)

    full = lambda s: pl.BlockSpec(s, lambda: (0,) * len(s))
    mu, logvar_sub, reconst, pred = pl.pallas_call(
        _stage2_body,
        in_specs=[
            full((B, L1)), full((L1, L2)), full((L1, L2)), full((L2, D)),
            full((D, D)), full((D, D)), full((D, D)), full((D, D)),
            full((L2, 1)), full((B, B)), full((B, D)),
        ],
        out_specs=[
            full((B, L2)), full((B, L2)), full((B, B)), full((2 * B, 1)),
        ],
        out_shape=[
            jax.ShapeDtypeStruct((B, L2), jnp.float32),
            jax.ShapeDtypeStruct((B, L2), jnp.float32),
            jax.ShapeDtypeStruct((B, B), jnp.float32),
            jax.ShapeDtypeStruct((2 * B, 1), jnp.float32),
        ],
    )(agg2, W2, W3, Wd1, Wd2, Wm1, Wm2, Wg, Wl, sub_adj,
      sub_feat)

    gan_label = jnp.concatenate(
        [jnp.ones((B, 1), jnp.float32), jnp.zeros((B, 1), jnp.float32)],
        axis=0,
    )
    return (mu, logvar_sub, reconst, pred, gan_label)


# speedup vs baseline: 1.0831x; 1.0831x over previous
import functools

import jax
import jax.numpy as jnp
import numpy as np
from jax import lax
from jax.experimental import pallas as pl
from jax.experimental.pallas import tpu as pltpu
from jax.experimental.pallas import tpu_sc as plsc

N = 10000
DEG = 32
D = 128
L1 = 256
L2 = 128
S = 25
B = 1024
BS = B * S
M = BS + B

NW = 32
RPW = BS // NW
NPW = B // NW
CR = 16
NCH = RPW // CR
CI = CR * S
CH = CI // 2
GG = 40
NG = CH // GG


def _tf_cipher(k1, k2, x0, x1):
    ks = [np.uint32(k1), np.uint32(k2), np.uint32(k1 ^ k2 ^ 0x1BD11BDA)]
    rots = [[13, 15, 26, 6], [17, 29, 16, 24]]
    x0 = (x0 + ks[0]).astype(np.uint32)
    x1 = (x1 + ks[1]).astype(np.uint32)
    for i in range(5):
        for d in rots[i % 2]:
            x0 = (x0 + x1).astype(np.uint32)
            x1 = ((x1 << np.uint32(d)) | (x1 >> np.uint32(32 - d))).astype(
                np.uint32
            )
            x1 = (x1 ^ x0).astype(np.uint32)
        x0 = (x0 + ks[(i + 1) % 3]).astype(np.uint32)
        x1 = (x1 + ks[(i + 2) % 3] + np.uint32(i + 1)).astype(np.uint32)
    return x0, x1


def _tf_bits(kd, size):
    o0, o1 = _tf_cipher(
        kd[0], kd[1], np.zeros(size, np.uint32),
        np.arange(size, dtype=np.uint32),
    )
    return (o0 ^ o1).astype(np.uint32)


def _tf_randint(kd, shape, maxval):
    size = int(np.prod(shape))
    o0, o1 = _tf_cipher(
        kd[0], kd[1], np.zeros(2, np.uint32), np.arange(2, dtype=np.uint32)
    )
    hi = _tf_bits((o0[0], o1[0]), size)
    lo = _tf_bits((o0[1], o1[1]), size)
    span = np.uint32(maxval)
    mult = np.uint32(((2**16 % maxval) ** 2) % maxval)
    off = ((hi % span) * mult + (lo % span)) % span
    return off.astype(np.int32).reshape(shape)


_s0, _s1 = _tf_cipher(
    np.uint32(0), np.uint32(42), np.zeros(2, np.uint32),
    np.arange(2, dtype=np.uint32),
)
_C2 = _tf_randint((_s0[0], _s1[0]), (B, S), DEG)
_C1 = _tf_randint((_s0[1], _s1[1]), (M, S), DEG)[:BS]
_C2R = _C2.reshape(NW, RPW // 16, 16)
_GMAT = np.zeros((32, 800), np.float32)
for _g in range(32):
    _GMAT[_g, _g * S:(_g + 1) * S] = 1.0 / S
_C1R = _C1.reshape(NW * NCH, CI // 16, 16)


def _sc_gather_body(adj_h, feat_h, nodes_h, c2_h, c1_h, agg1_h, subf_h,
                    nodes_v, adja_v, c2_v, sn_v, adjb0_v, adjb1_v, c10_v,
                    c11_v, idx0_v, idx1_v, fbuf0_v, fbuf1_v, acc_v, sf_v,
                    sem_a, sem_sf, sem_b0, sem_b1, sem_c10, sem_c11,
                    sem_g0, sem_g1):
    w = lax.axis_index("s") * 2 + lax.axis_index("c")
    inv_s = jnp.full((16,), 1.0 / S, dtype=jnp.float32)
    adjb = (adjb0_v, adjb1_v)
    c1b = (c10_v, c11_v)
    idxb = (idx0_v, idx1_v)
    fbuf = (fbuf0_v, fbuf1_v)
    sem_b = (sem_b0, sem_b1)
    sem_c1 = (sem_c10, sem_c11)
    sem_g = (sem_g0, sem_g1)

    pltpu.sync_copy(nodes_h.at[pl.ds(w * NPW, NPW)], nodes_v)
    cp_a = pltpu.async_copy(adj_h.at[nodes_v], adja_v, sem_a)
    cp_sf = pltpu.async_copy(feat_h.at[nodes_v], sf_v, sem_sf)
    pltpu.sync_copy(c2_h.at[w], c2_v)
    cp_a.wait()

    def sn_step(i, _):
        p = lax.iota(jnp.int32, 16) + i * 16
        rows = lax.shift_right_logical(p * 5243, 17)
        cols = c2_v[i, :]
        sn_v[i, :] = plsc.load_gather(adja_v, [rows, cols])
        return 0

    lax.fori_loop(0, RPW // 16, sn_step, 0)
    cp_sf.wait()
    pltpu.sync_copy(sf_v, subf_h.at[pl.ds(w * NPW, NPW)])

    def issue_pre(c, p):
        pltpu.async_copy(adj_h.at[sn_v.at[c]], adjb[p], sem_b[p])
        pltpu.async_copy(c1_h.at[w * NCH + c], c1b[p], sem_c1[p])

    def wait_pre(c, p):
        pltpu.make_async_copy(adj_h.at[sn_v.at[c]], adjb[p], sem_b[p]).wait()
        pltpu.make_async_copy(c1_h.at[w * NCH + c], c1b[p], sem_c1[p]).wait()

    def compute_idx(p):
        def idx_step(i, _):
            q = lax.iota(jnp.int32, 16) + i * 16
            rows = lax.shift_right_logical(q * 5243, 17)
            cols = c1b[p][i, :]
            idxb[p][pl.ds(i * 16, 16)] = plsc.load_gather(adjb[p], [rows, cols])
            return 0

        lax.fori_loop(0, CI // 16, idx_step, 0)

    def fire_half(ip, h):
        for j in range(NG):
            pltpu.async_copy(
                feat_h.at[idxb[ip].at[pl.ds(h * CH + j * GG, GG)]],
                fbuf[h].at[pl.ds(j * GG, GG)],
                sem_g[h],
            )

    def drain_half(h):
        pltpu.make_async_copy(feat_h.at[pl.ds(0, CH)], fbuf[h], sem_g[h]).wait()

    def accumulate_half(h, row0):
        fb = fbuf[h]

        def row_step(r, _):
            base = r * S
            acc = [fb[base, pl.ds(k * 16, 16)] for k in range(8)]
            for s in range(1, S):
                for k in range(8):
                    acc[k] = acc[k] + fb[base + s, pl.ds(k * 16, 16)]
            for k in range(8):
                acc_v[row0 + r, pl.ds(k * 16, 16)] = acc[k] * inv_s
            return 0

        lax.fori_loop(0, CR // 2, row_step, 0)

    pltpu.sync_copy(c1_h.at[w * NCH], c10_v)
    pltpu.async_copy(adj_h.at[sn_v.at[0]], adjb0_v, sem_b0).wait()
    compute_idx(0)
    fire_half(0, 0)
    issue_pre(1, 1)

    def pair(t, _):
        for pc in (0, 1):
            c = 2 * t + pc
            nxt, pp = c + 1, 1 - pc
            drain_half(0)
            fire_half(pc, 1)
            if pc == 0:
                wait_pre(nxt, pp)
                compute_idx(pp)

                @pl.when(t < (NCH // 2) - 1)
                def _():
                    issue_pre(nxt + 1, pc)
            else:

                @pl.when(t < (NCH // 2) - 1)
                def _():
                    wait_pre(nxt, pp)
                    compute_idx(pp)
                    issue_pre(nxt + 1, pc)

            accumulate_half(0, 0)
            drain_half(1)
            if pc == 0:
                fire_half(pp, 0)
            else:

                @pl.when(t < (NCH // 2) - 1)
                def _():
                    fire_half(pp, 0)

            accumulate_half(1, CR // 2)
            pltpu.sync_copy(acc_v, agg1_h.at[pl.ds(w * RPW + c * CR, CR)])
        return 0

    lax.fori_loop(0, NCH // 2, pair, 0)


@functools.partial(jax.jit, static_argnums=())
def _sc_gather(adj, features, nodes, c2r, c1r):
    mesh = plsc.VectorSubcoreMesh(core_axis_name="c", subcore_axis_name="s")
    return pl.kernel(
        _sc_gather_body,
        out_type=[
            jax.ShapeDtypeStruct((BS, D), jnp.float32),
            jax.ShapeDtypeStruct((B, D), jnp.float32),
        ],
        mesh=mesh,
        compiler_params=pltpu.CompilerParams(needs_layout_passes=False),
        scratch_types=[
            pltpu.VMEM((NPW,), jnp.int32),
            pltpu.VMEM((NPW, 128), jnp.int32),
            pltpu.VMEM((RPW // 16, 16), jnp.int32),
            pltpu.VMEM((NCH, CR), jnp.int32),
            pltpu.VMEM((CR, 128), jnp.int32),
            pltpu.VMEM((CR, 128), jnp.int32),
            pltpu.VMEM((CI // 16, 16), jnp.int32),
            pltpu.VMEM((CI // 16, 16), jnp.int32),
            pltpu.VMEM((CI,), jnp.int32),
            pltpu.VMEM((CI,), jnp.int32),
            pltpu.VMEM((CH, D), jnp.float32),
            pltpu.VMEM((CH, D), jnp.float32),
            pltpu.VMEM((CR, D), jnp.float32),
            pltpu.VMEM((NPW, D), jnp.float32),
            pltpu.SemaphoreType.DMA,
            pltpu.SemaphoreType.DMA,
            pltpu.SemaphoreType.DMA,
            pltpu.SemaphoreType.DMA,
            pltpu.SemaphoreType.DMA,
            pltpu.SemaphoreType.DMA,
            pltpu.SemaphoreType.DMA,
            pltpu.SemaphoreType.DMA,
        ],
    )(adj, features, nodes, c2r, c1r)


_R1 = 800
_G1 = _R1 // S
_HI = jax.lax.Precision.HIGHEST


def _l2n(x):
    n = jnp.sqrt(jnp.sum(x * x, axis=-1, keepdims=True))
    return x / jnp.maximum(n, 1e-12)


def _dot(a, b):
    return jnp.dot(a.astype(jnp.bfloat16), b.astype(jnp.bfloat16),
                   preferred_element_type=jnp.float32)


def _dot_hi(a, b):
    return jnp.dot(a, b, preferred_element_type=jnp.float32, precision=_HI)


def _dot_nt(a, b):
    return lax.dot_general(
        a.astype(jnp.bfloat16), b.astype(jnp.bfloat16),
        (((1,), (1,)), ((), ())), preferred_element_type=jnp.float32,
    )


def _stage1_body(agg1_ref, w1_ref, g_ref, out_ref):
    x = jnp.maximum(
        jnp.dot(agg1_ref[...], w1_ref[...],
                preferred_element_type=jnp.float32),
        0.0,
    )
    out_ref[...] = _dot_hi(g_ref[...], x)


def _stage2_body(agg2_ref, w2_ref, w3_ref, wd1_ref, wd2_ref, wm1_ref,
                 wm2_ref, wg_ref, wl_ref, sub_adj_ref, sub_feat_ref,
                 mu_ref, logv_ref, rec_ref, pred_ref):
    a = agg2_ref[...]
    mu = _dot(a, w2_ref[...])
    mu_ref[...] = mu
    logv_ref[...] = -_dot(a, w3_ref[...])
    h = _l2n(mu)
    o = _dot(h, wd1_ref[...])
    o = jnp.maximum(_l2n(o), 0.0)
    o = _dot(o, wd2_ref[...])
    o = jnp.maximum(_l2n(o), 0.0)
    e1 = _l2n(_dot(o, wm1_ref[...]))
    e2 = _l2n(_dot(o, wm2_ref[...]))
    rec = _dot_nt(e1, e2)
    rec_ref[...] = rec
    sg = _dot(sub_feat_ref[...], wg_ref[...])
    og = _dot(sub_adj_ref[...], sg)
    orig = _dot(og, wl_ref[...])
    ge = _dot(rec, sg)
    gen = _dot(ge, wl_ref[...])
    pred_ref[...] = jnp.concatenate([orig, gen], axis=0)


def kernel(nodes, sub_adj, adj, features, W1, W2, W3, Wd1, Wd2, Wm1, Wm2,
           Wg, Wl):
    nodes = nodes.astype(jnp.int32)
    adj_p = jnp.pad(adj.astype(jnp.int32), ((0, 0), (0, 128 - DEG)))
    agg1, sub_feat = _sc_gather(
        adj_p, features, nodes, jnp.asarray(_C2R), jnp.asarray(_C1R)
    )

    agg2 = pl.pallas_call(
        _stage1_body,
        grid=(BS // _R1,),
        in_specs=[
            pl.BlockSpec((_R1, D), lambda i: (i, 0)),
            pl.BlockSpec((D, L1), lambda i: (0, 0)),
            pl.BlockSpec((_G1, _R1), lambda i: (0, 0)),
        ],
        out_specs=pl.BlockSpec((_G1, L1), lambda i: (i, 0)),
        out_shape=jax.ShapeDtypeStruct((B, L1), jnp.float32),
    )(agg1.astype(jnp.bfloat16), W1.astype(jnp.bfloat16),
      jnp.asarray(_GMAT))

    full = lambda s: pl.BlockSpec(s, lambda: (0,) * len(s))
    mu, logvar_sub, reconst, pred = pl.pallas_call(
        _stage2_body,
        in_specs=[
            full((B, L1)), full((L1, L2)), full((L1, L2)), full((L2, D)),
            full((D, D)), full((D, D)), full((D, D)), full((D, D)),
            full((L2, 1)), full((B, B)), full((B, D)),
        ],
        out_specs=[
            full((B, L2)), full((B, L2)), full((B, B)), full((2 * B, 1)),
        ],
        out_shape=[
            jax.ShapeDtypeStruct((B, L2), jnp.float32),
            jax.ShapeDtypeStruct((B, L2), jnp.float32),
            jax.ShapeDtypeStruct((B, B), jnp.float32),
            jax.ShapeDtypeStruct((2 * B, 1), jnp.float32),
        ],
    )(agg2, W2, W3, Wd1, Wd2, Wm1, Wm2, Wg, Wl, sub_adj,
      sub_feat)

    gan_label = jnp.concatenate(
        [jnp.ones((B, 1), jnp.float32), jnp.zeros((B, 1), jnp.float32)],
        axis=0,
    )
    return (mu, logvar_sub, reconst, pred, gan_label)


# speedup vs baseline: 1.1041x; 1.0194x over previous
import functools

import jax
import jax.numpy as jnp
import numpy as np
from jax import lax
from jax.experimental import pallas as pl
from jax.experimental.pallas import tpu as pltpu
from jax.experimental.pallas import tpu_sc as plsc

N = 10000
DEG = 32
D = 128
L1 = 256
L2 = 128
S = 25
B = 1024
BS = B * S
M = BS + B

NW = 32
RPW = BS // NW
NPW = B // NW
CR = 16
NCH = RPW // CR
CI = CR * S
CH = CI // 2
GG = 40
NG = CH // GG


def _tf_cipher(k1, k2, x0, x1):
    ks = [np.uint32(k1), np.uint32(k2), np.uint32(k1 ^ k2 ^ 0x1BD11BDA)]
    rots = [[13, 15, 26, 6], [17, 29, 16, 24]]
    x0 = (x0 + ks[0]).astype(np.uint32)
    x1 = (x1 + ks[1]).astype(np.uint32)
    for i in range(5):
        for d in rots[i % 2]:
            x0 = (x0 + x1).astype(np.uint32)
            x1 = ((x1 << np.uint32(d)) | (x1 >> np.uint32(32 - d))).astype(
                np.uint32
            )
            x1 = (x1 ^ x0).astype(np.uint32)
        x0 = (x0 + ks[(i + 1) % 3]).astype(np.uint32)
        x1 = (x1 + ks[(i + 2) % 3] + np.uint32(i + 1)).astype(np.uint32)
    return x0, x1


def _tf_bits(kd, size):
    o0, o1 = _tf_cipher(
        kd[0], kd[1], np.zeros(size, np.uint32),
        np.arange(size, dtype=np.uint32),
    )
    return (o0 ^ o1).astype(np.uint32)


def _tf_randint(kd, shape, maxval):
    size = int(np.prod(shape))
    o0, o1 = _tf_cipher(
        kd[0], kd[1], np.zeros(2, np.uint32), np.arange(2, dtype=np.uint32)
    )
    hi = _tf_bits((o0[0], o1[0]), size)
    lo = _tf_bits((o0[1], o1[1]), size)
    span = np.uint32(maxval)
    mult = np.uint32(((2**16 % maxval) ** 2) % maxval)
    off = ((hi % span) * mult + (lo % span)) % span
    return off.astype(np.int32).reshape(shape)


_s0, _s1 = _tf_cipher(
    np.uint32(0), np.uint32(42), np.zeros(2, np.uint32),
    np.arange(2, dtype=np.uint32),
)
_C2 = _tf_randint((_s0[0], _s1[0]), (B, S), DEG)
_C1 = _tf_randint((_s0[1], _s1[1]), (M, S), DEG)[:BS]
_C2R = _C2.reshape(NW, RPW // 16, 16)
_GMAT = np.zeros((32, 800), np.float32)
for _g in range(32):
    _GMAT[_g, _g * S:(_g + 1) * S] = 1.0 / S
_C1R = _C1.reshape(NW * NCH, CI // 16, 16)


def _sc_gather_body(adj_h, feat_h, nodes_h, c2_h, c1_h, agg1_h, subf_h,
                    nodes_v, adja_v, c2_v, sn_v, adjb0_v, adjb1_v, c10_v,
                    c11_v, idx0_v, idx1_v, fbuf0_v, fbuf1_v, acc_v, sf_v,
                    sem_a, sem_sf, sem_b0, sem_b1, sem_c10, sem_c11,
                    sem_g0, sem_g1):
    w = lax.axis_index("s") * 2 + lax.axis_index("c")
    inv_s = jnp.full((16,), 1.0 / S, dtype=jnp.float32)
    adjb = (adjb0_v, adjb1_v)
    c1b = (c10_v, c11_v)
    idxb = (idx0_v, idx1_v)
    fbuf = (fbuf0_v, fbuf1_v)
    sem_b = (sem_b0, sem_b1)
    sem_c1 = (sem_c10, sem_c11)
    sem_g = (sem_g0, sem_g1)

    pltpu.sync_copy(nodes_h.at[pl.ds(w * NPW, NPW)], nodes_v)
    cp_a = pltpu.async_copy(adj_h.at[nodes_v], adja_v, sem_a)
    cp_sf = pltpu.async_copy(feat_h.at[nodes_v], sf_v, sem_sf)
    pltpu.sync_copy(c2_h.at[w], c2_v)
    cp_a.wait()

    def sn_step(i, _):
        p = lax.iota(jnp.int32, 16) + i * 16
        rows = lax.shift_right_logical(p * 5243, 17)
        cols = c2_v[i, :]
        sn_v[i, :] = plsc.load_gather(adja_v, [rows, cols])
        return 0

    lax.fori_loop(0, RPW // 16, sn_step, 0)
    cp_sf.wait()
    pltpu.sync_copy(sf_v, subf_h.at[pl.ds(w * NPW, NPW)])

    def issue_pre(c, p):
        pltpu.async_copy(adj_h.at[sn_v.at[c]], adjb[p], sem_b[p])
        pltpu.async_copy(c1_h.at[w * NCH + c], c1b[p], sem_c1[p])

    def wait_pre(c, p):
        pltpu.make_async_copy(adj_h.at[sn_v.at[c]], adjb[p], sem_b[p]).wait()
        pltpu.make_async_copy(c1_h.at[w * NCH + c], c1b[p], sem_c1[p]).wait()

    def compute_idx(p):
        def idx_step(i, _):
            q = lax.iota(jnp.int32, 16) + i * 16
            rows = lax.shift_right_logical(q * 5243, 17)
            cols = c1b[p][i, :]
            idxb[p][pl.ds(i * 16, 16)] = plsc.load_gather(adjb[p], [rows, cols])
            return 0

        lax.fori_loop(0, CI // 16, idx_step, 0)

    def fire_half(ip, h):
        for j in range(NG):
            pltpu.async_copy(
                feat_h.at[idxb[ip].at[pl.ds(h * CH + j * GG, GG)]],
                fbuf[h].at[pl.ds(j * GG, GG)],
                sem_g[h],
            )

    def drain_half(h):
        pltpu.make_async_copy(feat_h.at[pl.ds(0, CH)], fbuf[h], sem_g[h]).wait()

    def accumulate_half(h, row0):
        fb = fbuf[h]

        def row_step(r2, _):
            for u in range(2):
                base = (r2 * 2 + u) * S
                acc = [fb[base, pl.ds(k * 16, 16)] for k in range(8)]
                for s in range(1, S):
                    for k in range(8):
                        acc[k] = acc[k] + fb[base + s, pl.ds(k * 16, 16)]
                for k in range(8):
                    acc_v[row0 + r2 * 2 + u, pl.ds(k * 16, 16)] = (
                        acc[k] * inv_s
                    )
            return 0

        lax.fori_loop(0, CR // 4, row_step, 0)

    pltpu.sync_copy(c1_h.at[w * NCH], c10_v)
    pltpu.async_copy(adj_h.at[sn_v.at[0]], adjb0_v, sem_b0).wait()
    compute_idx(0)
    fire_half(0, 0)
    issue_pre(1, 1)

    def pair(t, _):
        for pc in (0, 1):
            c = 2 * t + pc
            nxt, pp = c + 1, 1 - pc
            drain_half(0)
            fire_half(pc, 1)
            if pc == 0:
                wait_pre(nxt, pp)
                compute_idx(pp)

                @pl.when(t < (NCH // 2) - 1)
                def _():
                    issue_pre(nxt + 1, pc)
            else:

                @pl.when(t < (NCH // 2) - 1)
                def _():
                    wait_pre(nxt, pp)
                    compute_idx(pp)
                    issue_pre(nxt + 1, pc)

            accumulate_half(0, 0)
            drain_half(1)
            if pc == 0:
                fire_half(pp, 0)
            else:

                @pl.when(t < (NCH // 2) - 1)
                def _():
                    fire_half(pp, 0)

            accumulate_half(1, CR // 2)
            pltpu.sync_copy(acc_v, agg1_h.at[pl.ds(w * RPW + c * CR, CR)])
        return 0

    lax.fori_loop(0, NCH // 2, pair, 0)


@functools.partial(jax.jit, static_argnums=())
def _sc_gather(adj, features, nodes, c2r, c1r):
    mesh = plsc.VectorSubcoreMesh(core_axis_name="c", subcore_axis_name="s")
    return pl.kernel(
        _sc_gather_body,
        out_type=[
            jax.ShapeDtypeStruct((BS, D), jnp.float32),
            jax.ShapeDtypeStruct((B, D), jnp.float32),
        ],
        mesh=mesh,
        compiler_params=pltpu.CompilerParams(needs_layout_passes=False),
        scratch_types=[
            pltpu.VMEM((NPW,), jnp.int32),
            pltpu.VMEM((NPW, 128), jnp.int32),
            pltpu.VMEM((RPW // 16, 16), jnp.int32),
            pltpu.VMEM((NCH, CR), jnp.int32),
            pltpu.VMEM((CR, 128), jnp.int32),
            pltpu.VMEM((CR, 128), jnp.int32),
            pltpu.VMEM((CI // 16, 16), jnp.int32),
            pltpu.VMEM((CI // 16, 16), jnp.int32),
            pltpu.VMEM((CI,), jnp.int32),
            pltpu.VMEM((CI,), jnp.int32),
            pltpu.VMEM((CH, D), jnp.float32),
            pltpu.VMEM((CH, D), jnp.float32),
            pltpu.VMEM((CR, D), jnp.float32),
            pltpu.VMEM((NPW, D), jnp.float32),
            pltpu.SemaphoreType.DMA,
            pltpu.SemaphoreType.DMA,
            pltpu.SemaphoreType.DMA,
            pltpu.SemaphoreType.DMA,
            pltpu.SemaphoreType.DMA,
            pltpu.SemaphoreType.DMA,
            pltpu.SemaphoreType.DMA,
            pltpu.SemaphoreType.DMA,
        ],
    )(adj, features, nodes, c2r, c1r)


_R1 = 800
_G1 = _R1 // S
_HI = jax.lax.Precision.HIGHEST


def _l2n(x):
    n = jnp.sqrt(jnp.sum(x * x, axis=-1, keepdims=True))
    return x / jnp.maximum(n, 1e-12)


def _dot(a, b):
    return jnp.dot(a.astype(jnp.bfloat16), b.astype(jnp.bfloat16),
                   preferred_element_type=jnp.float32)


def _dot_hi(a, b):
    return jnp.dot(a, b, preferred_element_type=jnp.float32, precision=_HI)


def _dot_nt(a, b):
    return lax.dot_general(
        a.astype(jnp.bfloat16), b.astype(jnp.bfloat16),
        (((1,), (1,)), ((), ())), preferred_element_type=jnp.float32,
    )


def _stage1_body(agg1_ref, w1_ref, g_ref, out_ref):
    x = jnp.maximum(_dot(agg1_ref[...], w1_ref[...]), 0.0)
    out_ref[...] = _dot_hi(g_ref[...], x)


def _stage2_body(agg2_ref, w2_ref, w3_ref, wd1_ref, wd2_ref, wm1_ref,
                 wm2_ref, wg_ref, wl_ref, sub_adj_ref, sub_feat_ref,
                 mu_ref, logv_ref, rec_ref, pred_ref):
    a = agg2_ref[...]
    mu = _dot(a, w2_ref[...])
    mu_ref[...] = mu
    logv_ref[...] = -_dot(a, w3_ref[...])
    h = _l2n(mu)
    o = _dot(h, wd1_ref[...])
    o = jnp.maximum(_l2n(o), 0.0)
    o = _dot(o, wd2_ref[...])
    o = jnp.maximum(_l2n(o), 0.0)
    e1 = _l2n(_dot(o, wm1_ref[...]))
    e2 = _l2n(_dot(o, wm2_ref[...]))
    rec = _dot_nt(e1, e2)
    rec_ref[...] = rec
    sg = _dot(sub_feat_ref[...], wg_ref[...])
    og = _dot(sub_adj_ref[...], sg)
    orig = _dot(og, wl_ref[...])
    ge = _dot(rec, sg)
    gen = _dot(ge, wl_ref[...])
    pred_ref[...] = jnp.concatenate([orig, gen], axis=0)


def kernel(nodes, sub_adj, adj, features, W1, W2, W3, Wd1, Wd2, Wm1, Wm2,
           Wg, Wl):
    nodes = nodes.astype(jnp.int32)
    adj_p = jnp.pad(adj.astype(jnp.int32), ((0, 0), (0, 128 - DEG)))
    agg1, sub_feat = _sc_gather(
        adj_p, features, nodes, jnp.asarray(_C2R), jnp.asarray(_C1R)
    )

    agg2 = pl.pallas_call(
        _stage1_body,
        grid=(BS // _R1,),
        in_specs=[
            pl.BlockSpec((_R1, D), lambda i: (i, 0)),
            pl.BlockSpec((D, L1), lambda i: (0, 0)),
            pl.BlockSpec((_G1, _R1), lambda i: (0, 0)),
        ],
        out_specs=pl.BlockSpec((_G1, L1), lambda i: (i, 0)),
        out_shape=jax.ShapeDtypeStruct((B, L1), jnp.float32),
    )(agg1, W1, jnp.asarray(_GMAT))

    full = lambda s: pl.BlockSpec(s, lambda: (0,) * len(s))
    mu, logvar_sub, reconst, pred = pl.pallas_call(
        _stage2_body,
        in_specs=[
            full((B, L1)), full((L1, L2)), full((L1, L2)), full((L2, D)),
            full((D, D)), full((D, D)), full((D, D)), full((D, D)),
            full((L2, 1)), full((B, B)), full((B, D)),
        ],
        out_specs=[
            full((B, L2)), full((B, L2)), full((B, B)), full((2 * B, 1)),
        ],
        out_shape=[
            jax.ShapeDtypeStruct((B, L2), jnp.float32),
            jax.ShapeDtypeStruct((B, L2), jnp.float32),
            jax.ShapeDtypeStruct((B, B), jnp.float32),
            jax.ShapeDtypeStruct((2 * B, 1), jnp.float32),
        ],
    )(agg2, W2, W3, Wd1, Wd2, Wm1, Wm2, Wg, Wl, sub_adj,
      sub_feat)

    gan_label = jnp.concatenate(
        [jnp.ones((B, 1), jnp.float32), jnp.zeros((B, 1), jnp.float32)],
        axis=0,
    )
    return (mu, logvar_sub, reconst, pred, gan_label)


# speedup vs baseline: 1.1131x; 1.0081x over previous
import functools

import jax
import jax.numpy as jnp
import numpy as np
from jax import lax
from jax.experimental import pallas as pl
from jax.experimental.pallas import tpu as pltpu
from jax.experimental.pallas import tpu_sc as plsc

N = 10000
DEG = 32
D = 128
L1 = 256
L2 = 128
S = 25
B = 1024
BS = B * S
M = BS + B

NW = 32
RPW = BS // NW
NPW = B // NW
CR = 16
NCH = RPW // CR
CI = CR * S
CH = CI // 2
GG = 40
NG = CH // GG


def _tf_cipher(k1, k2, x0, x1):
    ks = [np.uint32(k1), np.uint32(k2), np.uint32(k1 ^ k2 ^ 0x1BD11BDA)]
    rots = [[13, 15, 26, 6], [17, 29, 16, 24]]
    x0 = (x0 + ks[0]).astype(np.uint32)
    x1 = (x1 + ks[1]).astype(np.uint32)
    for i in range(5):
        for d in rots[i % 2]:
            x0 = (x0 + x1).astype(np.uint32)
            x1 = ((x1 << np.uint32(d)) | (x1 >> np.uint32(32 - d))).astype(
                np.uint32
            )
            x1 = (x1 ^ x0).astype(np.uint32)
        x0 = (x0 + ks[(i + 1) % 3]).astype(np.uint32)
        x1 = (x1 + ks[(i + 2) % 3] + np.uint32(i + 1)).astype(np.uint32)
    return x0, x1


def _tf_bits(kd, size):
    o0, o1 = _tf_cipher(
        kd[0], kd[1], np.zeros(size, np.uint32),
        np.arange(size, dtype=np.uint32),
    )
    return (o0 ^ o1).astype(np.uint32)


def _tf_randint(kd, shape, maxval):
    size = int(np.prod(shape))
    o0, o1 = _tf_cipher(
        kd[0], kd[1], np.zeros(2, np.uint32), np.arange(2, dtype=np.uint32)
    )
    hi = _tf_bits((o0[0], o1[0]), size)
    lo = _tf_bits((o0[1], o1[1]), size)
    span = np.uint32(maxval)
    mult = np.uint32(((2**16 % maxval) ** 2) % maxval)
    off = ((hi % span) * mult + (lo % span)) % span
    return off.astype(np.int32).reshape(shape)


_s0, _s1 = _tf_cipher(
    np.uint32(0), np.uint32(42), np.zeros(2, np.uint32),
    np.arange(2, dtype=np.uint32),
)
_C2 = _tf_randint((_s0[0], _s1[0]), (B, S), DEG)
_C1 = _tf_randint((_s0[1], _s1[1]), (M, S), DEG)[:BS]
_C2R = _C2.reshape(NW, RPW // 16, 16)
_C1R = _C1.reshape(NW * NCH, CI // 16, 16)


def _sc_gather_body(adj_h, feat_h, nodes_h, c2_h, c1_h, agg1_h, subf_h,
                    nodes_v, adja_v, c2_v, sn_v, adjb0_v, adjb1_v, c10_v,
                    c11_v, idx0_v, idx1_v, fbuf0_v, fbuf1_v, acc_v, sf_v,
                    sem_a, sem_sf, sem_b0, sem_b1, sem_c10, sem_c11,
                    sem_g0, sem_g1):
    w = lax.axis_index("s") * 2 + lax.axis_index("c")
    inv_s = jnp.full((16,), 1.0 / S, dtype=jnp.float32)
    adjb = (adjb0_v, adjb1_v)
    c1b = (c10_v, c11_v)
    idxb = (idx0_v, idx1_v)
    fbuf = (fbuf0_v, fbuf1_v)
    sem_b = (sem_b0, sem_b1)
    sem_c1 = (sem_c10, sem_c11)
    sem_g = (sem_g0, sem_g1)

    pltpu.sync_copy(nodes_h.at[pl.ds(w * NPW, NPW)], nodes_v)
    cp_a = pltpu.async_copy(adj_h.at[nodes_v], adja_v, sem_a)
    cp_sf = pltpu.async_copy(feat_h.at[nodes_v], sf_v, sem_sf)
    pltpu.sync_copy(c2_h.at[w], c2_v)
    cp_a.wait()

    def sn_step(i, _):
        p = lax.iota(jnp.int32, 16) + i * 16
        rows = lax.shift_right_logical(p * 5243, 17)
        cols = c2_v[i, :]
        sn_v[i, :] = plsc.load_gather(adja_v, [rows, cols])
        return 0

    lax.fori_loop(0, RPW // 16, sn_step, 0)
    cp_sf.wait()
    pltpu.sync_copy(sf_v, subf_h.at[pl.ds(w * NPW, NPW)])

    def issue_pre(c, p):
        pltpu.async_copy(adj_h.at[sn_v.at[c]], adjb[p], sem_b[p])
        pltpu.async_copy(c1_h.at[w * NCH + c], c1b[p], sem_c1[p])

    def wait_pre(c, p):
        pltpu.make_async_copy(adj_h.at[sn_v.at[c]], adjb[p], sem_b[p]).wait()
        pltpu.make_async_copy(c1_h.at[w * NCH + c], c1b[p], sem_c1[p]).wait()

    def compute_idx(p):
        def idx_step(i, _):
            q = lax.iota(jnp.int32, 16) + i * 16
            rows = lax.shift_right_logical(q * 5243, 17)
            cols = c1b[p][i, :]
            idxb[p][pl.ds(i * 16, 16)] = plsc.load_gather(adjb[p], [rows, cols])
            return 0

        lax.fori_loop(0, CI // 16, idx_step, 0)

    def fire_half(ip, h):
        for j in range(NG):
            pltpu.async_copy(
                feat_h.at[idxb[ip].at[pl.ds(h * CH + j * GG, GG)]],
                fbuf[h].at[pl.ds(j * GG, GG)],
                sem_g[h],
            )

    def drain_half(h):
        pltpu.make_async_copy(feat_h.at[pl.ds(0, CH)], fbuf[h], sem_g[h]).wait()

    def accumulate_half(h, row0):
        fb = fbuf[h]

        def row_step(r2, _):
            for u in range(2):
                base = (r2 * 2 + u) * S
                acc = [fb[base, pl.ds(k * 16, 16)] for k in range(8)]
                for s in range(1, S):
                    for k in range(8):
                        acc[k] = acc[k] + fb[base + s, pl.ds(k * 16, 16)]
                for k in range(8):
                    acc_v[row0 + r2 * 2 + u, pl.ds(k * 16, 16)] = (
                        acc[k] * inv_s
                    )
            return 0

        lax.fori_loop(0, CR // 4, row_step, 0)

    pltpu.sync_copy(c1_h.at[w * NCH], c10_v)
    pltpu.async_copy(adj_h.at[sn_v.at[0]], adjb0_v, sem_b0).wait()
    compute_idx(0)
    fire_half(0, 0)
    issue_pre(1, 1)

    def pair(t, _):
        for pc in (0, 1):
            c = 2 * t + pc
            nxt, pp = c + 1, 1 - pc
            drain_half(0)
            fire_half(pc, 1)
            if pc == 0:
                wait_pre(nxt, pp)
                compute_idx(pp)

                @pl.when(t < (NCH // 2) - 1)
                def _():
                    issue_pre(nxt + 1, pc)
            else:

                @pl.when(t < (NCH // 2) - 1)
                def _():
                    wait_pre(nxt, pp)
                    compute_idx(pp)
                    issue_pre(nxt + 1, pc)

            accumulate_half(0, 0)
            drain_half(1)
            if pc == 0:
                fire_half(pp, 0)
            else:

                @pl.when(t < (NCH // 2) - 1)
                def _():
                    fire_half(pp, 0)

            accumulate_half(1, CR // 2)
            pltpu.sync_copy(acc_v, agg1_h.at[pl.ds(w * RPW + c * CR, CR)])
        return 0

    lax.fori_loop(0, NCH // 2, pair, 0)


@functools.partial(jax.jit, static_argnums=())
def _sc_gather(adj, features, nodes, c2r, c1r):
    mesh = plsc.VectorSubcoreMesh(core_axis_name="c", subcore_axis_name="s")
    return pl.kernel(
        _sc_gather_body,
        out_type=[
            jax.ShapeDtypeStruct((BS, D), jnp.float32),
            jax.ShapeDtypeStruct((B, D), jnp.float32),
        ],
        mesh=mesh,
        compiler_params=pltpu.CompilerParams(needs_layout_passes=False),
        scratch_types=[
            pltpu.VMEM((NPW,), jnp.int32),
            pltpu.VMEM((NPW, 128), jnp.int32),
            pltpu.VMEM((RPW // 16, 16), jnp.int32),
            pltpu.VMEM((NCH, CR), jnp.int32),
            pltpu.VMEM((CR, 128), jnp.int32),
            pltpu.VMEM((CR, 128), jnp.int32),
            pltpu.VMEM((CI // 16, 16), jnp.int32),
            pltpu.VMEM((CI // 16, 16), jnp.int32),
            pltpu.VMEM((CI,), jnp.int32),
            pltpu.VMEM((CI,), jnp.int32),
            pltpu.VMEM((CH, D), jnp.float32),
            pltpu.VMEM((CH, D), jnp.float32),
            pltpu.VMEM((CR, D), jnp.float32),
            pltpu.VMEM((NPW, D), jnp.float32),
            pltpu.SemaphoreType.DMA,
            pltpu.SemaphoreType.DMA,
            pltpu.SemaphoreType.DMA,
            pltpu.SemaphoreType.DMA,
            pltpu.SemaphoreType.DMA,
            pltpu.SemaphoreType.DMA,
            pltpu.SemaphoreType.DMA,
            pltpu.SemaphoreType.DMA,
        ],
    )(adj, features, nodes, c2r, c1r)


_R1 = 800
_G1 = _R1 // S
_HI = jax.lax.Precision.HIGHEST


def _l2n(x):
    n = jnp.sqrt(jnp.sum(x * x, axis=-1, keepdims=True))
    return x / jnp.maximum(n, 1e-12)


def _dot(a, b):
    return jnp.dot(a.astype(jnp.bfloat16), b.astype(jnp.bfloat16),
                   preferred_element_type=jnp.float32)


def _dot_hi(a, b):
    return jnp.dot(a, b, preferred_element_type=jnp.float32, precision=_HI)


def _dot_nt(a, b):
    return lax.dot_general(
        a.astype(jnp.bfloat16), b.astype(jnp.bfloat16),
        (((1,), (1,)), ((), ())), preferred_element_type=jnp.float32,
    )


def _stage1_body(agg1_ref, w1_ref, out_ref):
    x = jnp.maximum(_dot(agg1_ref[...], w1_ref[...]), 0.0)
    out_ref[...] = jnp.sum(x.reshape(_G1, S, L1), axis=1) * (1.0 / S)


def _stage2_body(agg2_ref, w2_ref, w3_ref, wd1_ref, wd2_ref, wm1_ref,
                 wm2_ref, wg_ref, wl_ref, sub_adj_ref, sub_feat_ref,
                 mu_ref, logv_ref, rec_ref, pred_ref):
    a = agg2_ref[...]
    mu = _dot(a, w2_ref[...])
    mu_ref[...] = mu
    logv_ref[...] = -_dot(a, w3_ref[...])
    h = _l2n(mu)
    o = _dot(h, wd1_ref[...])
    o = jnp.maximum(_l2n(o), 0.0)
    o = _dot(o, wd2_ref[...])
    o = jnp.maximum(_l2n(o), 0.0)
    e1 = _l2n(_dot(o, wm1_ref[...]))
    e2 = _l2n(_dot(o, wm2_ref[...]))
    rec = _dot_nt(e1, e2)
    rec_ref[...] = rec
    sg = _dot(sub_feat_ref[...], wg_ref[...])
    og = _dot(sub_adj_ref[...], sg)
    orig = _dot(og, wl_ref[...])
    ge = _dot(rec, sg)
    gen = _dot(ge, wl_ref[...])
    pred_ref[...] = jnp.concatenate([orig, gen], axis=0)


def kernel(nodes, sub_adj, adj, features, W1, W2, W3, Wd1, Wd2, Wm1, Wm2,
           Wg, Wl):
    nodes = nodes.astype(jnp.int32)
    adj_p = jnp.pad(adj.astype(jnp.int32), ((0, 0), (0, 128 - DEG)))
    agg1, sub_feat = _sc_gather(
        adj_p, features, nodes, jnp.asarray(_C2R), jnp.asarray(_C1R)
    )

    agg2 = pl.pallas_call(
        _stage1_body,
        grid=(BS // _R1,),
        in_specs=[
            pl.BlockSpec((_R1, D), lambda i: (i, 0)),
            pl.BlockSpec((D, L1), lambda i: (0, 0)),
        ],
        out_specs=pl.BlockSpec((_G1, L1), lambda i: (i, 0)),
        out_shape=jax.ShapeDtypeStruct((B, L1), jnp.float32),
    )(agg1, W1)

    full = lambda s: pl.BlockSpec(s, lambda: (0,) * len(s))
    mu, logvar_sub, reconst, pred = pl.pallas_call(
        _stage2_body,
        in_specs=[
            full((B, L1)), full((L1, L2)), full((L1, L2)), full((L2, D)),
            full((D, D)), full((D, D)), full((D, D)), full((D, D)),
            full((L2, 1)), full((B, B)), full((B, D)),
        ],
        out_specs=[
            full((B, L2)), full((B, L2)), full((B, B)), full((2 * B, 1)),
        ],
        out_shape=[
            jax.ShapeDtypeStruct((B, L2), jnp.float32),
            jax.ShapeDtypeStruct((B, L2), jnp.float32),
            jax.ShapeDtypeStruct((B, B), jnp.float32),
            jax.ShapeDtypeStruct((2 * B, 1), jnp.float32),
        ],
    )(agg2, W2, W3, Wd1, Wd2, Wm1, Wm2, Wg, Wl, sub_adj,
      sub_feat)

    gan_label = jnp.concatenate(
        [jnp.ones((B, 1), jnp.float32), jnp.zeros((B, 1), jnp.float32)],
        axis=0,
    )
    return (mu, logvar_sub, reconst, pred, gan_label)
